# baseline (speedup 1.0000x reference)
import functools

import jax
import jax.numpy as jnp
from jax import lax
from jax.experimental import pallas as pl
from jax.experimental.pallas import tpu as pltpu
from jax.experimental.pallas import tpu_sc as plsc

VOCAB = 70873
EMBED_DIM = 128
BATCH = 4096
SEQ = 50

NC = 2
NS = 16
NW = NC * NS
B = BATCH * SEQ
BPW = B // NW
CHUNK = 128
NCH = BPW // CHUNK
BIG = 2 * CHUNK
NBC = BPW // BIG
NBUF = 3


def _emb_body(ids_hbm, table_hbm, out_hbm, idx_v, rows_v, gsem, ssem):
    wid = lax.axis_index("s") * NC + lax.axis_index("c")
    base = wid * BPW

    pltpu.sync_copy(ids_hbm.at[wid], idx_v)

    def start_gathers(J, b):
        pltpu.async_copy(
            table_hbm.at[idx_v.at[2 * J]], rows_v.at[b].at[pl.ds(0, CHUNK)], gsem
        )
        pltpu.async_copy(
            table_hbm.at[idx_v.at[2 * J + 1]],
            rows_v.at[b].at[pl.ds(CHUNK, CHUNK)],
            gsem,
        )

    def start_store(J, b):
        pltpu.async_copy(
            rows_v.at[b], out_hbm.at[pl.ds(base + J * BIG, BIG)], ssem
        )

    def wait_gathers(b):
        for k in range(2):
            pltpu.make_async_copy(
                table_hbm.at[idx_v.at[0]],
                rows_v.at[b].at[pl.ds(k * CHUNK, CHUNK)],
                gsem,
            ).wait()

    def wait_store(b):
        pltpu.make_async_copy(rows_v.at[b], out_hbm.at[pl.ds(0, BIG)], ssem).wait()

    for b in range(NBUF):
        start_gathers(b, b)

    @pl.loop(0, NBC - 1, step=NBUF)
    def _(J0):
        for db in range(NBUF):
            J = J0 + db
            wait_gathers(db)
            start_store(J, db)
            wait_store(db)

            @pl.when(J + NBUF < NBC)
            def _():
                start_gathers(J + NBUF, db)

    wait_gathers((NBC - 1) % NBUF)
    start_store(NBC - 1, (NBC - 1) % NBUF)
    wait_store((NBC - 1) % NBUF)


@functools.cache
def _build():
    mesh = plsc.VectorSubcoreMesh(core_axis_name="c", subcore_axis_name="s")
    return functools.partial(
        pl.kernel,
        mesh=mesh,
        out_type=jax.ShapeDtypeStruct((B, EMBED_DIM), jnp.float32),
        scratch_types=[
            pltpu.VMEM((NCH, CHUNK), jnp.int32),
            pltpu.VMEM((NBUF, BIG, EMBED_DIM), jnp.float32),
            pltpu.SemaphoreType.DMA,
            pltpu.SemaphoreType.DMA,
        ],
    )(_emb_body)


def kernel(input_ids, table):
    ids = input_ids.T.reshape(NW, NCH, CHUNK).astype(jnp.int32)
    out = _build()(ids, table)
    return out.reshape(SEQ, BATCH, EMBED_DIM).transpose(1, 0, 2)
